# Initial kernel scaffold; baseline (speedup 1.0000x reference)
import jax
import jax.numpy as jnp
from jax.experimental import pallas as pl


def kernel(feats, edge_emb, cos_edge_emb, W_prep0, W_prep1, W_edge_prep, W_e0, W_e1, W_n0, W_n1, W_fc, b_fc, node2edge_idx, edge_node_adj, cos_node2edge_idx, cos_edge_node_adj, train_ids):
    raise NotImplementedError("write your pallas kernel here")


# speedup vs baseline: 5.4703x; 5.4703x over previous
import functools

import jax
import jax.numpy as jnp
from jax import lax
from jax.experimental import pallas as pl
from jax.experimental.pallas import tpu as pltpu
from jax.experimental.pallas import tpu_sc as plsc

N = 10000
E = 320000
S = 32
DF = 128
DE = 16
P = 32
C = 8
B = 1024

NC = 2
NS = 16
NW = NC * NS
EPT = E // NW
NPAD = 10240
NPT = NPAD // NW
CH = 128
ECH = EPT // CH
ETAIL = EPT - ECH * CH
NCH = (NPT * S) // CH

_f32 = jnp.float32


def _edge_phase(wid, x0_hbm, idx0, idx1, a0_hbm, a1_hbm, msum_hbm,
                bufA, bufB, mbuf, sA, sB, sW):
    ebase = wid * EPT
    pltpu.sync_copy(a0_hbm.at[pl.ds(ebase, EPT)], idx0)
    pltpu.sync_copy(a1_hbm.at[pl.ds(ebase, EPT)], idx1)

    def fire(j, b):
        pltpu.make_async_copy(
            x0_hbm.at[idx0.at[pl.ds(j * CH, CH)]], bufA[b], sA[b]).start()
        pltpu.make_async_copy(
            x0_hbm.at[idx1.at[pl.ds(j * CH, CH)]], bufB[b], sB[b]).start()

    fire(0, 0)
    fire(1, 1)

    @pl.loop(0, ECH, step=2)
    def _(j0):
        for b in (0, 1):
            j = j0 + b
            pltpu.make_async_copy(
                x0_hbm.at[idx0.at[pl.ds(0, CH)]], bufA[b], sA[b]).wait()
            pltpu.make_async_copy(
                x0_hbm.at[idx1.at[pl.ds(0, CH)]], bufB[b], sB[b]).wait()

            @pl.when(j >= 2)
            def _():
                pltpu.make_async_copy(
                    mbuf[b], msum_hbm.at[pl.ds(0, CH)], sW[b]).wait()

            @pl.loop(0, CH, unroll=8)
            def _(r):
                mbuf[b][r, pl.ds(0, 16)] = (
                    bufA[b][r, pl.ds(0, 16)] + bufB[b][r, pl.ds(0, 16)])
                mbuf[b][r, pl.ds(16, 16)] = (
                    bufA[b][r, pl.ds(16, 16)] + bufB[b][r, pl.ds(16, 16)])

            @pl.when(j + 2 < ECH)
            def _():
                fire(j + 2, b)

            pltpu.make_async_copy(
                mbuf[b], msum_hbm.at[pl.ds(ebase + j * CH, CH)], sW[b]).start()

    pltpu.make_async_copy(mbuf[0], msum_hbm.at[pl.ds(0, CH)], sW[0]).wait()
    pltpu.make_async_copy(mbuf[1], msum_hbm.at[pl.ds(0, CH)], sW[1]).wait()

    toff = ECH * CH
    pltpu.make_async_copy(
        x0_hbm.at[idx0.at[pl.ds(toff, ETAIL)]],
        bufA[0].at[pl.ds(0, ETAIL)], sA[0]).start()
    pltpu.make_async_copy(
        x0_hbm.at[idx1.at[pl.ds(toff, ETAIL)]],
        bufB[0].at[pl.ds(0, ETAIL)], sB[0]).start()
    pltpu.make_async_copy(
        x0_hbm.at[idx0.at[pl.ds(toff, ETAIL)]],
        bufA[0].at[pl.ds(0, ETAIL)], sA[0]).wait()
    pltpu.make_async_copy(
        x0_hbm.at[idx1.at[pl.ds(toff, ETAIL)]],
        bufB[0].at[pl.ds(0, ETAIL)], sB[0]).wait()

    @pl.loop(0, ETAIL, unroll=8)
    def _(r):
        mbuf[0][r, pl.ds(0, 16)] = (
            bufA[0][r, pl.ds(0, 16)] + bufB[0][r, pl.ds(0, 16)])
        mbuf[0][r, pl.ds(16, 16)] = (
            bufA[0][r, pl.ds(16, 16)] + bufB[0][r, pl.ds(16, 16)])

    pltpu.sync_copy(mbuf[0].at[pl.ds(0, ETAIL)],
                    msum_hbm.at[pl.ds(ebase + toff, ETAIL)])


def _node_phase(wid, tbl_hbm, idxn, n2e_hbm, otile, out_hbm, width,
                gbuf, sG):
    nbase = wid * (NPT * S)
    pltpu.sync_copy(n2e_hbm.at[pl.ds(nbase, NPT * S)], idxn)

    def fire(j, b):
        pltpu.make_async_copy(
            tbl_hbm.at[idxn.at[pl.ds(j * CH, CH)]], gbuf[b], sG[b]).start()

    fire(0, 0)
    fire(1, 1)

    nodes_per_ch = CH // S

    @pl.loop(0, NCH, step=2)
    def _(j0):
        for b in (0, 1):
            j = j0 + b
            pltpu.make_async_copy(
                tbl_hbm.at[idxn.at[pl.ds(0, CH)]], gbuf[b], sG[b]).wait()

            @pl.loop(0, nodes_per_ch)
            def _(k):
                base = k * S
                acc0 = gbuf[b][base, pl.ds(0, 16)]
                if width == 32:
                    acc1 = gbuf[b][base, pl.ds(16, 16)]
                for r in range(1, S):
                    acc0 = acc0 + gbuf[b][base + r, pl.ds(0, 16)]
                    if width == 32:
                        acc1 = acc1 + gbuf[b][base + r, pl.ds(16, 16)]
                row = j * nodes_per_ch + k
                otile[row, pl.ds(0, 16)] = acc0
                if width == 32:
                    otile[row, pl.ds(16, 16)] = acc1

            @pl.when(j + 2 < NCH)
            def _():
                fire(j + 2, b)

    pltpu.sync_copy(otile, out_hbm.at[pl.ds(wid * NPT, NPT)])


def _sc1_body(x0_hbm, ee_a_hbm, ee_b_hbm, a0a_hbm, a1a_hbm, a0b_hbm, a1b_hbm,
              n2ea_hbm, n2eb_hbm,
              msum_a_hbm, msum_b_hbm, mesum_a_hbm, mesum_b_hbm,
              idx0, idx1, idxn, bufA0, bufA1, bufB0, bufB1, mbuf0, mbuf1,
              gbuf0, gbuf1, metile,
              sA0, sA1, sB0, sB1, sW0, sW1, sG0, sG1):
    wid = lax.axis_index("s") * NC + lax.axis_index("c")
    bufA, bufB, mbuf = (bufA0, bufA1), (bufB0, bufB1), (mbuf0, mbuf1)
    sA, sB, sW, sG = (sA0, sA1), (sB0, sB1), (sW0, sW1), (sG0, sG1)

    _edge_phase(wid, x0_hbm, idx0, idx1, a0a_hbm, a1a_hbm, msum_a_hbm,
                bufA, bufB, mbuf, sA, sB, sW)
    _edge_phase(wid, x0_hbm, idx0, idx1, a0b_hbm, a1b_hbm, msum_b_hbm,
                bufA, bufB, mbuf, sA, sB, sW)
    _node_phase(wid, ee_a_hbm, idxn, n2ea_hbm, metile, mesum_a_hbm, DE,
                (gbuf0, gbuf1), sG)
    _node_phase(wid, ee_b_hbm, idxn, n2eb_hbm, metile, mesum_b_hbm, DE,
                (gbuf0, gbuf1), sG)


@functools.lru_cache(maxsize=None)
def _sc1_build():
  return functools.partial(
    pl.kernel,
    mesh=plsc.VectorSubcoreMesh(core_axis_name="c", subcore_axis_name="s", num_cores=NC, num_subcores=NS),
    out_type=[
        jax.ShapeDtypeStruct((E, P), _f32),
        jax.ShapeDtypeStruct((E, P), _f32),
        jax.ShapeDtypeStruct((NPAD, DE), _f32),
        jax.ShapeDtypeStruct((NPAD, DE), _f32),
    ],
    scratch_types=[
        pltpu.VMEM((EPT,), jnp.int32),
        pltpu.VMEM((EPT,), jnp.int32),
        pltpu.VMEM((NPT * S,), jnp.int32),
        pltpu.VMEM((CH, P), _f32),
        pltpu.VMEM((CH, P), _f32),
        pltpu.VMEM((CH, P), _f32),
        pltpu.VMEM((CH, P), _f32),
        pltpu.VMEM((CH, P), _f32),
        pltpu.VMEM((CH, P), _f32),
        pltpu.VMEM((CH, DE), _f32),
        pltpu.VMEM((CH, DE), _f32),
        pltpu.VMEM((NPT, DE), _f32),
    ] + [pltpu.SemaphoreType.DMA] * 8,
    compiler_params=pltpu.CompilerParams(use_tc_tiling_on_sc=False),
  )(_sc1_body)


def _sc1(*args):
    return _sc1_build()(*args)


def _sc2_body(e1a_hbm, e1b_hbm, n2ea_hbm, n2eb_hbm,
              g1a_hbm, g1b_hbm,
              idxn, gbuf0, gbuf1, gtile, sG0, sG1):
    wid = lax.axis_index("s") * NC + lax.axis_index("c")
    _node_phase(wid, e1a_hbm, idxn, n2ea_hbm, gtile, g1a_hbm, P,
                (gbuf0, gbuf1), (sG0, sG1))
    _node_phase(wid, e1b_hbm, idxn, n2eb_hbm, gtile, g1b_hbm, P,
                (gbuf0, gbuf1), (sG0, sG1))


@functools.lru_cache(maxsize=None)
def _sc2_build():
  return functools.partial(
    pl.kernel,
    mesh=plsc.VectorSubcoreMesh(core_axis_name="c", subcore_axis_name="s", num_cores=NC, num_subcores=NS),
    out_type=[
        jax.ShapeDtypeStruct((NPAD, P), _f32),
        jax.ShapeDtypeStruct((NPAD, P), _f32),
    ],
    scratch_types=[
        pltpu.VMEM((NPT * S,), jnp.int32),
        pltpu.VMEM((CH, P), _f32),
        pltpu.VMEM((CH, P), _f32),
        pltpu.VMEM((NPT, P), _f32),
    ] + [pltpu.SemaphoreType.DMA] * 2,
    compiler_params=pltpu.CompilerParams(use_tc_tiling_on_sc=False),
  )(_sc2_body)


def _sc2(*args):
    return _sc2_build()(*args)


def _tc_prep_body(feats_ref, wp0_ref, wp1_ref, x0_ref, dummy_ref):
    f = feats_ref[...]
    x0_ref[...] = jnp.dot(f, wp1_ref[...], preferred_element_type=_f32)
    dummy_ref[...] = jnp.dot(f, wp0_ref[...], preferred_element_type=_f32)


def _tc_prep(feats, wp0, wp1):
    return pl.pallas_call(
        _tc_prep_body,
        out_shape=[jax.ShapeDtypeStruct((N, P), _f32)] * 2,
    )(feats, wp0, wp1)


EBLK = 4000


def _tc_mid_body(eea_ref, msa_ref, eeb_ref, msb_ref, wep_ref, we0_ref,
                 e1a_ref, e1b_ref):
    wc = jnp.dot(wep_ref[...], we0_ref[0:P, :], preferred_element_type=_f32)
    wb = we0_ref[P:2 * P, :] * 0.5
    e1a_ref[...] = jnp.maximum(
        jnp.dot(eea_ref[...], wc, preferred_element_type=_f32)
        + jnp.dot(msa_ref[...], wb, preferred_element_type=_f32), 0.0)
    e1b_ref[...] = jnp.maximum(
        jnp.dot(eeb_ref[...], wc, preferred_element_type=_f32)
        + jnp.dot(msb_ref[...], wb, preferred_element_type=_f32), 0.0)


def _tc_mid(ee_a, msum_a, ee_b, msum_b, wep, we0):
    nblk = E // EBLK
    espec = pl.BlockSpec((EBLK, DE), lambda i: (i, 0))
    mspec = pl.BlockSpec((EBLK, P), lambda i: (i, 0))
    wspec_ep = pl.BlockSpec((DE, P), lambda i: (0, 0))
    wspec_e0 = pl.BlockSpec((2 * P, P), lambda i: (0, 0))
    ospec = pl.BlockSpec((EBLK, P), lambda i: (i, 0))
    return pl.pallas_call(
        _tc_mid_body,
        grid=(nblk,),
        in_specs=[espec, mspec, espec, mspec, wspec_ep, wspec_e0],
        out_specs=[ospec, ospec],
        out_shape=[jax.ShapeDtypeStruct((E, P), _f32)] * 2,
        compiler_params=pltpu.CompilerParams(
            dimension_semantics=("arbitrary",)),
    )(ee_a, msum_a, ee_b, msum_b, wep, we0)


def _tc_final_body(dummy_ref, wep_ref, wn0_ref, wn1_ref, wfc_ref, bfc_ref,
                   mea_ref, meb_ref, g1a_ref, g1b_ref, tids_ref,
                   preds_ref, loss_ref):
    dummy = dummy_ref[...]
    wn0t = wn0_ref[0:P, :]
    wn0b = wn0_ref[P:2 * P, :]
    wn1t = wn1_ref[0:P, :]
    wn1b = wn1_ref[P:2 * P, :] * (1.0 / S)
    w2 = jnp.dot(wep_ref[...] * (1.0 / S), wn0b, preferred_element_type=_f32)

    d_top = jnp.dot(dummy, wn0t, preferred_element_type=_f32)
    x1a = jnp.maximum(
        d_top + jnp.dot(mea_ref[...], w2, preferred_element_type=_f32), 0.0)
    x1b = jnp.maximum(
        d_top + jnp.dot(meb_ref[...], w2, preferred_element_type=_f32), 0.0)
    x2a = jnp.maximum(
        jnp.dot(x1a, wn1t, preferred_element_type=_f32)
        + jnp.dot(g1a_ref[...], wn1b, preferred_element_type=_f32), 0.0)
    x2b = jnp.maximum(
        jnp.dot(x1b, wn1t, preferred_element_type=_f32)
        + jnp.dot(g1b_ref[...], wn1b, preferred_element_type=_f32), 0.0)

    h = jnp.concatenate([x1a, x2a], axis=1)
    hc = jnp.concatenate([x1b, x2b], axis=1)

    dot = jnp.sum(h * hc, axis=1)
    nh = jnp.sqrt(jnp.sum(h * h, axis=1)) + 1e-8
    nhc = jnp.sqrt(jnp.sum(hc * hc, axis=1)) + 1e-8
    loss_ref[...] = jnp.reshape(
        1.0 - jnp.sum(dot / (nh * nhc)) * (1.0 / N), (1, 1))

    logits = jnp.dot(h, wfc_ref[...], preferred_element_type=_f32) \
        + bfc_ref[...]
    tid = tids_ref[...]
    acc = jnp.zeros((B, C), _f32)
    blk = 1000
    for c0 in range(0, N, blk):
        ids = lax.broadcasted_iota(jnp.int32, (B, blk), 1) + c0
        oh = (tid == ids).astype(_f32)
        acc = acc + jnp.dot(oh, logits[c0:c0 + blk, :],
                            preferred_element_type=_f32)
    preds_ref[...] = acc


def _tc_final(dummy, wep, wn0, wn1, wfc, bfc2d, mea, meb, g1a, g1b, tids2d):
    return pl.pallas_call(
        _tc_final_body,
        out_shape=[
            jax.ShapeDtypeStruct((B, C), _f32),
            jax.ShapeDtypeStruct((1, 1), _f32),
        ],
    )(dummy, wep, wn0, wn1, wfc, bfc2d, mea, meb, g1a, g1b, tids2d)


def kernel(feats, edge_emb, cos_edge_emb, W_prep0, W_prep1, W_edge_prep,
           W_e0, W_e1, W_n0, W_n1, W_fc, b_fc, node2edge_idx, edge_node_adj,
           cos_node2edge_idx, cos_edge_node_adj, train_ids):
    a0a = edge_node_adj[:, 0]
    a1a = edge_node_adj[:, 1]
    a0b = cos_edge_node_adj[:, 0]
    a1b = cos_edge_node_adj[:, 1]
    n2ea = jnp.pad(node2edge_idx, ((0, NPAD - N), (0, 0))).reshape(-1)
    n2eb = jnp.pad(cos_node2edge_idx, ((0, NPAD - N), (0, 0))).reshape(-1)

    x0, dummy = _tc_prep(feats, W_prep0, W_prep1)

    msum_a, msum_b, mesum_a, mesum_b = _sc1(
        x0, edge_emb, cos_edge_emb, a0a, a1a, a0b, a1b, n2ea, n2eb)

    e1a, e1b = _tc_mid(edge_emb, msum_a, cos_edge_emb, msum_b,
                       W_edge_prep, W_e0)

    g1a, g1b = _sc2(e1a, e1b, n2ea, n2eb)

    preds, loss = _tc_final(
        dummy, W_edge_prep, W_n0, W_n1, W_fc, b_fc.reshape(1, C),
        mesum_a[:N], mesum_b[:N], g1a[:N], g1b[:N],
        train_ids.reshape(B, 1).astype(jnp.int32))
    return preds, loss[0, 0]


# speedup vs baseline: 9.1276x; 1.6686x over previous
import functools

import jax
import jax.numpy as jnp
from jax import lax
from jax.experimental import pallas as pl
from jax.experimental.pallas import tpu as pltpu
from jax.experimental.pallas import tpu_sc as plsc

N = 10000
E = 320000
S = 32
DF = 128
DE = 16
P = 32
C = 8
B = 1024

NC = 2
NS = 16
NW = NC * NS
EPT = E // NW
NPAD = 10240
NPT = NPAD // NW
CH = 128
ECH = EPT // CH
ETAIL = EPT - ECH * CH
NCH = (NPT * S) // CH

_f32 = jnp.float32


def _edge_phase(wid, x0_hbm, idx0, idx1, a0_hbm, a1_hbm, msum_hbm,
                bufA, bufB, mbuf, sA, sB, sW):
    ebase = wid * EPT
    pbase = wid * (EPT // 4)
    PCH = CH // 4
    pltpu.sync_copy(a0_hbm.at[pl.ds(ebase, EPT)], idx0)
    pltpu.sync_copy(a1_hbm.at[pl.ds(ebase, EPT)], idx1)

    def fire(j, b):
        pltpu.make_async_copy(
            x0_hbm.at[idx0.at[pl.ds(j * CH, CH)]], bufA[b], sA[b]).start()
        pltpu.make_async_copy(
            x0_hbm.at[idx1.at[pl.ds(j * CH, CH)]], bufB[b], sB[b]).start()

    fire(0, 0)
    fire(1, 1)

    @pl.loop(0, ECH, step=2)
    def _(j0):
        for b in (0, 1):
            j = j0 + b
            pltpu.make_async_copy(
                x0_hbm.at[idx0.at[pl.ds(0, CH)]], bufA[b], sA[b]).wait()
            pltpu.make_async_copy(
                x0_hbm.at[idx1.at[pl.ds(0, CH)]], bufB[b], sB[b]).wait()

            @pl.when(j >= 2)
            def _():
                pltpu.make_async_copy(
                    mbuf[b], msum_hbm.at[pl.ds(0, PCH)], sW[b]).wait()

            @pl.loop(0, CH, unroll=8)
            def _(r):
                pr = r // 4
                pc = (r % 4) * P
                mbuf[b][pr, pl.ds(pc, 16)] = (
                    bufA[b][r, pl.ds(0, 16)] + bufB[b][r, pl.ds(0, 16)])
                mbuf[b][pr, pl.ds(pc + 16, 16)] = (
                    bufA[b][r, pl.ds(16, 16)] + bufB[b][r, pl.ds(16, 16)])

            @pl.when(j + 2 < ECH)
            def _():
                fire(j + 2, b)

            pltpu.make_async_copy(
                mbuf[b], msum_hbm.at[pl.ds(pbase + j * PCH, PCH)],
                sW[b]).start()

    pltpu.make_async_copy(mbuf[0], msum_hbm.at[pl.ds(0, PCH)], sW[0]).wait()
    pltpu.make_async_copy(mbuf[1], msum_hbm.at[pl.ds(0, PCH)], sW[1]).wait()

    toff = ECH * CH
    pltpu.make_async_copy(
        x0_hbm.at[idx0.at[pl.ds(toff, ETAIL)]],
        bufA[0].at[pl.ds(0, ETAIL)], sA[0]).start()
    pltpu.make_async_copy(
        x0_hbm.at[idx1.at[pl.ds(toff, ETAIL)]],
        bufB[0].at[pl.ds(0, ETAIL)], sB[0]).start()
    pltpu.make_async_copy(
        x0_hbm.at[idx0.at[pl.ds(toff, ETAIL)]],
        bufA[0].at[pl.ds(0, ETAIL)], sA[0]).wait()
    pltpu.make_async_copy(
        x0_hbm.at[idx1.at[pl.ds(toff, ETAIL)]],
        bufB[0].at[pl.ds(0, ETAIL)], sB[0]).wait()

    @pl.loop(0, ETAIL, unroll=8)
    def _(r):
        pr = r // 4
        pc = (r % 4) * P
        mbuf[0][pr, pl.ds(pc, 16)] = (
            bufA[0][r, pl.ds(0, 16)] + bufB[0][r, pl.ds(0, 16)])
        mbuf[0][pr, pl.ds(pc + 16, 16)] = (
            bufA[0][r, pl.ds(16, 16)] + bufB[0][r, pl.ds(16, 16)])

    pltpu.sync_copy(mbuf[0].at[pl.ds(0, ETAIL // 4)],
                    msum_hbm.at[pl.ds(pbase + toff // 4, ETAIL // 4)])


def _node_phase(wid, tbl_hbm, idxn, n2e_hbm, otile, out_hbm, width,
                gbuf, sG):
    nbase = wid * (NPT * S)
    pltpu.sync_copy(n2e_hbm.at[pl.ds(nbase, NPT * S)], idxn)

    def fire(j, b):
        pltpu.make_async_copy(
            tbl_hbm.at[idxn.at[pl.ds(j * CH, CH)]], gbuf[b], sG[b]).start()

    fire(0, 0)
    fire(1, 1)

    nodes_per_ch = CH // S

    @pl.loop(0, NCH, step=2)
    def _(j0):
        for b in (0, 1):
            j = j0 + b
            pltpu.make_async_copy(
                tbl_hbm.at[idxn.at[pl.ds(0, CH)]], gbuf[b], sG[b]).wait()

            @pl.loop(0, nodes_per_ch)
            def _(k):
                base = k * S
                acc0 = gbuf[b][base, pl.ds(0, 16)]
                if width == 32:
                    acc1 = gbuf[b][base, pl.ds(16, 16)]
                for r in range(1, S):
                    acc0 = acc0 + gbuf[b][base + r, pl.ds(0, 16)]
                    if width == 32:
                        acc1 = acc1 + gbuf[b][base + r, pl.ds(16, 16)]
                row = j * nodes_per_ch + k
                otile[row, pl.ds(0, 16)] = acc0
                if width == 32:
                    otile[row, pl.ds(16, 16)] = acc1

            @pl.when(j + 2 < NCH)
            def _():
                fire(j + 2, b)

    pltpu.sync_copy(otile, out_hbm.at[pl.ds(wid * NPT, NPT)])


def _sc1_body(x0_hbm, ee_a_hbm, ee_b_hbm, a0a_hbm, a1a_hbm, a0b_hbm, a1b_hbm,
              n2ea_hbm, n2eb_hbm,
              msum_a_hbm, msum_b_hbm, mesum_a_hbm, mesum_b_hbm,
              idx0, idx1, idxn, bufA0, bufA1, bufB0, bufB1, mbuf0, mbuf1,
              gbuf0, gbuf1, metile,
              sA0, sA1, sB0, sB1, sW0, sW1, sG0, sG1):
    wid = lax.axis_index("s") * NC + lax.axis_index("c")
    bufA, bufB, mbuf = (bufA0, bufA1), (bufB0, bufB1), (mbuf0, mbuf1)
    sA, sB, sW, sG = (sA0, sA1), (sB0, sB1), (sW0, sW1), (sG0, sG1)

    _edge_phase(wid, x0_hbm, idx0, idx1, a0a_hbm, a1a_hbm, msum_a_hbm,
                bufA, bufB, mbuf, sA, sB, sW)
    _edge_phase(wid, x0_hbm, idx0, idx1, a0b_hbm, a1b_hbm, msum_b_hbm,
                bufA, bufB, mbuf, sA, sB, sW)
    _node_phase(wid, ee_a_hbm, idxn, n2ea_hbm, metile, mesum_a_hbm, DE,
                (gbuf0, gbuf1), sG)
    _node_phase(wid, ee_b_hbm, idxn, n2eb_hbm, metile, mesum_b_hbm, DE,
                (gbuf0, gbuf1), sG)


@functools.lru_cache(maxsize=None)
def _sc1_build():
  return functools.partial(
    pl.kernel,
    mesh=plsc.VectorSubcoreMesh(core_axis_name="c", subcore_axis_name="s", num_cores=NC, num_subcores=NS),
    out_type=[
        jax.ShapeDtypeStruct((E // 4, 4 * P), _f32),
        jax.ShapeDtypeStruct((E // 4, 4 * P), _f32),
        jax.ShapeDtypeStruct((NPAD, DE), _f32),
        jax.ShapeDtypeStruct((NPAD, DE), _f32),
    ],
    scratch_types=[
        pltpu.VMEM((EPT,), jnp.int32),
        pltpu.VMEM((EPT,), jnp.int32),
        pltpu.VMEM((NPT * S,), jnp.int32),
        pltpu.VMEM((CH, P), _f32),
        pltpu.VMEM((CH, P), _f32),
        pltpu.VMEM((CH, P), _f32),
        pltpu.VMEM((CH, P), _f32),
        pltpu.VMEM((CH // 4, 4 * P), _f32),
        pltpu.VMEM((CH // 4, 4 * P), _f32),
        pltpu.VMEM((CH, DE), _f32),
        pltpu.VMEM((CH, DE), _f32),
        pltpu.VMEM((NPT, DE), _f32),
    ] + [pltpu.SemaphoreType.DMA] * 8,
    compiler_params=pltpu.CompilerParams(use_tc_tiling_on_sc=False),
  )(_sc1_body)


def _sc1(*args):
    return _sc1_build()(*args)


def _sc2_body(e1a_hbm, e1b_hbm, n2ea_hbm, n2eb_hbm,
              g1a_hbm, g1b_hbm,
              idxn, gbuf0, gbuf1, gtile, sG0, sG1):
    wid = lax.axis_index("s") * NC + lax.axis_index("c")
    _node_phase(wid, e1a_hbm, idxn, n2ea_hbm, gtile, g1a_hbm, P,
                (gbuf0, gbuf1), (sG0, sG1))
    _node_phase(wid, e1b_hbm, idxn, n2eb_hbm, gtile, g1b_hbm, P,
                (gbuf0, gbuf1), (sG0, sG1))


@functools.lru_cache(maxsize=None)
def _sc2_build():
  return functools.partial(
    pl.kernel,
    mesh=plsc.VectorSubcoreMesh(core_axis_name="c", subcore_axis_name="s", num_cores=NC, num_subcores=NS),
    out_type=[
        jax.ShapeDtypeStruct((NPAD, P), _f32),
        jax.ShapeDtypeStruct((NPAD, P), _f32),
    ],
    scratch_types=[
        pltpu.VMEM((NPT * S,), jnp.int32),
        pltpu.VMEM((CH, P), _f32),
        pltpu.VMEM((CH, P), _f32),
        pltpu.VMEM((NPT, P), _f32),
    ] + [pltpu.SemaphoreType.DMA] * 2,
    compiler_params=pltpu.CompilerParams(use_tc_tiling_on_sc=False),
  )(_sc2_body)


def _sc2(*args):
    return _sc2_build()(*args)


def _tc_prep_body(feats_ref, wp0_ref, wp1_ref, x0_ref, dummy_ref):
    f = feats_ref[...]
    x0_ref[...] = jnp.dot(f, wp1_ref[...], preferred_element_type=_f32)
    dummy_ref[...] = jnp.dot(f, wp0_ref[...], preferred_element_type=_f32)


def _tc_prep(feats, wp0, wp1):
    return pl.pallas_call(
        _tc_prep_body,
        out_shape=[jax.ShapeDtypeStruct((N, P), _f32)] * 2,
    )(feats, wp0, wp1)


EBLK4 = 4000


def _tc_mid_body(eep_a_ref, msp_a_ref, eep_b_ref, msp_b_ref,
                 kep_ref, ke0t_ref, ke0b_ref, e1a_ref, e1b_ref):
    wkc = jnp.dot(kep_ref[...], ke0t_ref[...], preferred_element_type=_f32)
    wkb = ke0b_ref[...] * 0.5
    e1a_ref[...] = jnp.maximum(
        jnp.dot(eep_a_ref[...], wkc, preferred_element_type=_f32)
        + jnp.dot(msp_a_ref[...], wkb, preferred_element_type=_f32), 0.0)
    e1b_ref[...] = jnp.maximum(
        jnp.dot(eep_b_ref[...], wkc, preferred_element_type=_f32)
        + jnp.dot(msp_b_ref[...], wkb, preferred_element_type=_f32), 0.0)


def _tc_mid(eep_a, msp_a, eep_b, msp_b, kep, ke0t, ke0b):
    nblk = (E // 4) // EBLK4
    espec = pl.BlockSpec((EBLK4, 4 * DE), lambda i: (i, 0))
    mspec = pl.BlockSpec((EBLK4, 4 * P), lambda i: (i, 0))
    wspec_ep = pl.BlockSpec((4 * DE, 4 * P), lambda i: (0, 0))
    wspec_e0 = pl.BlockSpec((4 * P, 4 * P), lambda i: (0, 0))
    ospec = pl.BlockSpec((EBLK4, 4 * P), lambda i: (i, 0))
    return pl.pallas_call(
        _tc_mid_body,
        grid=(nblk,),
        in_specs=[espec, mspec, espec, mspec, wspec_ep, wspec_e0, wspec_e0],
        out_specs=[ospec, ospec],
        out_shape=[jax.ShapeDtypeStruct((E // 4, 4 * P), _f32)] * 2,
        compiler_params=pltpu.CompilerParams(
            dimension_semantics=("arbitrary",)),
    )(eep_a, msp_a, eep_b, msp_b, kep, ke0t, ke0b)


def _tc_final_body(dummy_ref, wep_ref, wn0_ref, wn1_ref, wfc_ref, bfc_ref,
                   mea_ref, meb_ref, g1a_ref, g1b_ref, tids_ref,
                   preds_ref, loss_ref):
    dummy = dummy_ref[...]
    wn0t = wn0_ref[0:P, :]
    wn0b = wn0_ref[P:2 * P, :]
    wn1t = wn1_ref[0:P, :]
    wn1b = wn1_ref[P:2 * P, :] * (1.0 / S)
    w2 = jnp.dot(wep_ref[...] * (1.0 / S), wn0b, preferred_element_type=_f32)

    d_top = jnp.dot(dummy, wn0t, preferred_element_type=_f32)
    x1a = jnp.maximum(
        d_top + jnp.dot(mea_ref[...], w2, preferred_element_type=_f32), 0.0)
    x1b = jnp.maximum(
        d_top + jnp.dot(meb_ref[...], w2, preferred_element_type=_f32), 0.0)
    x2a = jnp.maximum(
        jnp.dot(x1a, wn1t, preferred_element_type=_f32)
        + jnp.dot(g1a_ref[...], wn1b, preferred_element_type=_f32), 0.0)
    x2b = jnp.maximum(
        jnp.dot(x1b, wn1t, preferred_element_type=_f32)
        + jnp.dot(g1b_ref[...], wn1b, preferred_element_type=_f32), 0.0)

    h = jnp.concatenate([x1a, x2a], axis=1)
    hc = jnp.concatenate([x1b, x2b], axis=1)

    dot = jnp.sum(h * hc, axis=1)
    nh = jnp.sqrt(jnp.sum(h * h, axis=1)) + 1e-8
    nhc = jnp.sqrt(jnp.sum(hc * hc, axis=1)) + 1e-8
    loss_ref[...] = jnp.reshape(
        1.0 - jnp.sum(dot / (nh * nhc)) * (1.0 / N), (1, 1))

    logits = jnp.dot(h, wfc_ref[...], preferred_element_type=_f32) \
        + bfc_ref[...]
    tid = tids_ref[...]
    acc = jnp.zeros((B, C), _f32)
    blk = 1000
    for c0 in range(0, N, blk):
        ids = lax.broadcasted_iota(jnp.int32, (B, blk), 1) + c0
        oh = (tid == ids).astype(_f32)
        acc = acc + jnp.dot(oh, logits[c0:c0 + blk, :],
                            preferred_element_type=_f32)
    preds_ref[...] = acc


def _tc_final(dummy, wep, wn0, wn1, wfc, bfc2d, mea, meb, g1a, g1b, tids2d):
    return pl.pallas_call(
        _tc_final_body,
        out_shape=[
            jax.ShapeDtypeStruct((B, C), _f32),
            jax.ShapeDtypeStruct((1, 1), _f32),
        ],
    )(dummy, wep, wn0, wn1, wfc, bfc2d, mea, meb, g1a, g1b, tids2d)


def kernel(feats, edge_emb, cos_edge_emb, W_prep0, W_prep1, W_edge_prep,
           W_e0, W_e1, W_n0, W_n1, W_fc, b_fc, node2edge_idx, edge_node_adj,
           cos_node2edge_idx, cos_edge_node_adj, train_ids):
    a0a = edge_node_adj[:, 0]
    a1a = edge_node_adj[:, 1]
    a0b = cos_edge_node_adj[:, 0]
    a1b = cos_edge_node_adj[:, 1]
    n2ea = jnp.pad(node2edge_idx, ((0, NPAD - N), (0, 0))).reshape(-1)
    n2eb = jnp.pad(cos_node2edge_idx, ((0, NPAD - N), (0, 0))).reshape(-1)

    eep_a = jnp.reshape(edge_emb, (E // 4, 4 * DE))
    eep_b = jnp.reshape(cos_edge_emb, (E // 4, 4 * DE))
    eye4 = jnp.eye(4, dtype=_f32)
    kep = jnp.kron(eye4, W_edge_prep)
    ke0t = jnp.kron(eye4, W_e0[:P, :])
    ke0b = jnp.kron(eye4, W_e0[P:, :])

    x0, dummy = _tc_prep(feats, W_prep0, W_prep1)

    msp_a, msp_b, mesum_a, mesum_b = _sc1(
        x0, edge_emb, cos_edge_emb, a0a, a1a, a0b, a1b, n2ea, n2eb)

    e1a_p, e1b_p = _tc_mid(eep_a, msp_a, eep_b, msp_b, kep, ke0t, ke0b)

    g1a, g1b = _sc2(jnp.reshape(e1a_p, (E, P)), jnp.reshape(e1b_p, (E, P)),
                    n2ea, n2eb)

    preds, loss = _tc_final(
        dummy, W_edge_prep, W_n0, W_n1, W_fc, b_fc.reshape(1, C),
        mesum_a[:N], mesum_b[:N], g1a[:N], g1b[:N],
        train_ids.reshape(B, 1).astype(jnp.int32))
    return preds, loss[0, 0]
